# baseline (speedup 1.0000x reference)
import functools

import jax
import jax.numpy as jnp
from jax import lax
from jax.experimental import pallas as pl
from jax.experimental.pallas import tpu as pltpu
from jax.experimental.pallas import tpu_sc as plsc

_NUM_CLASSES = 1000
_BATCH = 16384
_NC = 1
_NS = 16
_NW = _NC * _NS
_ROWS_PER_W = _BATCH // _NW
_CHUNK = 32
_N_CHUNKS = _ROWS_PER_W // _CHUNK
_GROUPS = _CHUNK // 16
_FULL = _NUM_CLASSES // 16
_TAIL = _NUM_CLASSES - _FULL * 16


def _sc_body(label_hbm, out_hbm, label_v, buf_a, buf_b, lsem, sem_a, sem_b):
    wid = lax.axis_index("s") * _NC + lax.axis_index("c")
    row0 = wid * _ROWS_PER_W
    lcopy = pltpu.make_async_copy(
        label_hbm.at[pl.ds(row0, _ROWS_PER_W)], label_v, lsem
    )
    lcopy.start()

    zeros = jnp.zeros((16,), jnp.int32)
    ones = jnp.ones((16,), jnp.int32)
    iota = lax.iota(jnp.int32, 16)
    tail_col = _FULL * 16 + iota
    tail_mask = iota < _TAIL
    bufs = (buf_a, buf_b)
    sems = (sem_a, sem_b)

    def zero_body(r, carry):
        row_splat = jnp.full((16,), 0, jnp.int32) + r
        for buf in bufs:
            for g in range(_FULL):
                buf[r, pl.ds(g * 16, 16)] = zeros
            plsc.store_scatter(
                buf, [row_splat, tail_col], zeros, mask=tail_mask
            )
        return carry

    lax.fori_loop(0, _CHUNK, zero_body, 0)
    lcopy.wait()

    prev = [None, None]
    copies = [None, None]
    for c in range(_N_CHUNKS):
        b = c % 2
        buf = bufs[b]
        if copies[b] is not None:
            copies[b].wait()
            for row16, lv in prev[b]:
                plsc.store_scatter(buf, [row16, lv], zeros)
        idxs = []
        for g in range(_GROUPS):
            lv = label_v[pl.ds(c * _CHUNK + g * 16, 16)]
            row16 = g * 16 + iota
            idxs.append((row16, lv))
            plsc.store_scatter(buf, [row16, lv], ones)
        cp = pltpu.make_async_copy(
            buf, out_hbm.at[pl.ds(row0 + c * _CHUNK, _CHUNK)], sems[b]
        )
        cp.start()
        copies[b] = cp
        prev[b] = idxs
    copies[(_N_CHUNKS - 2) % 2].wait()
    copies[(_N_CHUNKS - 1) % 2].wait()


_one_hot_sc = functools.partial(
    pl.kernel,
    out_type=jax.ShapeDtypeStruct((_BATCH, _NUM_CLASSES), jnp.int32),
    mesh=plsc.VectorSubcoreMesh(core_axis_name="c", subcore_axis_name="s", num_cores=1),
    compiler_params=pltpu.CompilerParams(needs_layout_passes=False),
    scratch_types=[
        pltpu.VMEM((_ROWS_PER_W,), jnp.int32),
        pltpu.VMEM((_CHUNK, _NUM_CLASSES), jnp.int32),
        pltpu.VMEM((_CHUNK, _NUM_CLASSES), jnp.int32),
        pltpu.SemaphoreType.DMA,
        pltpu.SemaphoreType.DMA,
        pltpu.SemaphoreType.DMA,
    ],
)(_sc_body)


def kernel(label):
    return _one_hot_sc(label)


# speedup vs baseline: 2.5180x; 2.5180x over previous
import functools

import jax
import jax.numpy as jnp
from jax import lax
from jax.experimental import pallas as pl
from jax.experimental.pallas import tpu as pltpu
from jax.experimental.pallas import tpu_sc as plsc

_NUM_CLASSES = 1000
_BATCH = 16384
_NC = 2
_NS = 16
_NW = _NC * _NS
_COLS_PER_W = _BATCH // _NW
_CCHUNK = 200
_N_CHUNKS = _NUM_CLASSES // _CCHUNK
_GROUPS = _COLS_PER_W // 16
_BUF_WORDS = _CCHUNK * _COLS_PER_W


def _sc_body(label_hbm, out_hbm, label_v, buf_v, lsem):
    wid = lax.axis_index("s") * _NC + lax.axis_index("c")
    col0 = wid * _COLS_PER_W
    lcopy = pltpu.make_async_copy(
        label_hbm.at[pl.ds(col0, _COLS_PER_W)], label_v, lsem
    )
    lcopy.start()

    zeros = jnp.zeros((16,), jnp.int32)
    ones = jnp.ones((16,), jnp.int32)
    iota = lax.iota(jnp.int32, 16)

    def zero_body(r, carry):
        for g in range(_GROUPS):
            buf_v[r, pl.ds(g * 16, 16)] = zeros
        return carry

    lax.fori_loop(0, _CCHUNK, zero_body, 0)
    lcopy.wait()

    for c in range(_N_CHUNKS):
        c0 = c * _CCHUNK
        idxs = []
        for g in range(_GROUPS):
            lv = label_v[pl.ds(g * 16, 16)]
            row = lv - c0
            col = g * 16 + iota
            mask = (lv >= c0) & (lv < c0 + _CCHUNK)
            idxs.append((row, col, mask))
            plsc.store_scatter(buf_v, [row, col], ones, mask=mask)
        pltpu.sync_copy(
            buf_v,
            out_hbm.at[pl.ds(c0, _CCHUNK), pl.ds(col0, _COLS_PER_W)],
        )
        if c < _N_CHUNKS - 1:
            for row, col, mask in idxs:
                plsc.store_scatter(buf_v, [row, col], zeros, mask=mask)


_one_hot_sc_t = functools.partial(
    pl.kernel,
    out_type=jax.ShapeDtypeStruct((_NUM_CLASSES, _BATCH), jnp.int32),
    mesh=plsc.VectorSubcoreMesh(core_axis_name="c", subcore_axis_name="s"),
    compiler_params=pltpu.CompilerParams(needs_layout_passes=False),
    scratch_types=[
        pltpu.VMEM((_COLS_PER_W,), jnp.int32),
        pltpu.VMEM((_CCHUNK, _COLS_PER_W), jnp.int32),
        pltpu.SemaphoreType.DMA,
    ],
)(_sc_body)


def kernel(label):
    return _one_hot_sc_t(label).T
